# Initial kernel scaffold; baseline (speedup 1.0000x reference)
import jax
import jax.numpy as jnp
from jax.experimental import pallas as pl


def kernel(x, edge_index, batch, W1a, b1a, W1b, b1b, W2a, b2a, W2b, b2b, W3a, b3a, W3b, b3b, Wta, bta, Wtb, btb, Wtc, btc, Wr, br, Wc, bc):
    raise NotImplementedError("write your pallas kernel here")


# speedup vs baseline: 5.6153x; 5.6153x over previous
import functools

import jax
import jax.numpy as jnp
from jax import lax
from jax.experimental import pallas as pl
from jax.experimental.pallas import tpu as pltpu
from jax.experimental.pallas import tpu_sc as plsc

N = 10000
E = 320000
H = 128
C = 2

NC = 2
NS = 16
NW = NC * NS
CHUNK = 128
NCHUNKS = E // CHUNK
STRIPE = 624
TAIL_BASE = NS * STRIPE
TAIL = N - TAIL_BASE

_sc_mesh = plsc.VectorSubcoreMesh(core_axis_name="c", subcore_axis_name="s")


@functools.partial(
    pl.kernel,
    out_type=jax.ShapeDtypeStruct((NC, N, H), jnp.float32),
    mesh=_sc_mesh,
    scratch_types=[
        pltpu.VMEM((CHUNK,), jnp.int32),
        pltpu.VMEM((CHUNK,), jnp.int32),
        pltpu.VMEM((CHUNK, H), jnp.float32),
        pltpu.VMEM_SHARED((N, H), jnp.float32),
        pltpu.SemaphoreType.DMA,
    ],
)
def _sc_aggregate(h_hbm, src_hbm, dst_hbm, zeros_hbm, out_hbm,
                  src_v, dst_v, rows_v, agg_sh, sem):
    c = lax.axis_index("c")
    s = lax.axis_index("s")
    wid = s * NC + c

    base = s * STRIPE
    pltpu.sync_copy(zeros_hbm.at[pl.ds(base, STRIPE)],
                    agg_sh.at[pl.ds(base, STRIPE)])

    @pl.when(s == NS - 1)
    def _():
        pltpu.sync_copy(zeros_hbm.at[pl.ds(TAIL_BASE, TAIL)],
                        agg_sh.at[pl.ds(TAIL_BASE, TAIL)])

    plsc.subcore_barrier()

    nchunks_w = (NCHUNKS - wid + NW - 1) // NW

    def body(j, carry):
        ebase = (wid + j * NW) * CHUNK
        pltpu.sync_copy(src_hbm.at[pl.ds(ebase, CHUNK)], src_v)
        pltpu.sync_copy(dst_hbm.at[pl.ds(ebase, CHUNK)], dst_v)
        pltpu.async_copy(h_hbm.at[src_v], rows_v, sem).wait()
        pltpu.sync_copy(rows_v, agg_sh.at[dst_v], add=True)
        return carry

    lax.fori_loop(0, nchunks_w, body, 0)
    plsc.subcore_barrier()

    pltpu.sync_copy(agg_sh.at[pl.ds(base, STRIPE)],
                    out_hbm.at[c, pl.ds(base, STRIPE)])

    @pl.when(s == NS - 1)
    def _():
        pltpu.sync_copy(agg_sh.at[pl.ds(TAIL_BASE, TAIL)],
                        out_hbm.at[c, pl.ds(TAIL_BASE, TAIL)])


def _tc_layer_body(h_ref, p_ref, wa_ref, ba_ref, wb_ref, bb_ref, out_ref):
    z = h_ref[...] + p_ref[0] + p_ref[1]
    z = jnp.dot(z, wa_ref[...], preferred_element_type=jnp.float32) + ba_ref[...]
    z = jnp.maximum(z, 0.0)
    y = jnp.dot(z, wb_ref[...], preferred_element_type=jnp.float32) + bb_ref[...]
    out_ref[...] = jnp.maximum(y, 0.0)


_tc_layer = pl.pallas_call(
    _tc_layer_body,
    out_shape=jax.ShapeDtypeStruct((N, H), jnp.float32),
)


def _tc_attn_body(x_ref, wta_ref, bta_ref, wtb_ref, btb_ref, wtc_ref, btc_ref,
                  wr_ref, br_ref, wc_ref, bc_ref, out_ref):
    x = x_ref[...]
    a = jnp.tanh(jnp.dot(x, wta_ref[...], preferred_element_type=jnp.float32)
                 + bta_ref[...])
    g = jax.nn.sigmoid(jnp.dot(x, wtb_ref[...], preferred_element_type=jnp.float32)
                       + btb_ref[...])
    s = jnp.dot(a * g, wtc_ref[...], preferred_element_type=jnp.float32) + btc_ref[...]
    s = s[:, :1]
    m = jnp.max(s)
    e = jnp.exp(s - m)
    l = jnp.sum(e)
    hp = jnp.sum(e * x, axis=0, keepdims=True) / l
    h = jnp.maximum(jnp.dot(hp, wr_ref[...], preferred_element_type=jnp.float32)
                    + br_ref[...], 0.0)
    lg = jnp.dot(h, wc_ref[...], preferred_element_type=jnp.float32) + bc_ref[...]
    out_ref[...] = lg


_tc_attn = pl.pallas_call(
    _tc_attn_body,
    out_shape=jax.ShapeDtypeStruct((1, H), jnp.float32),
)


def kernel(x, edge_index, batch, W1a, b1a, W1b, b1b, W2a, b2a, W2b, b2b,
           W3a, b3a, W3b, b3b, Wta, bta, Wtb, btb, Wtc, btc, Wr, br, Wc, bc):
    src = edge_index[0]
    dst = edge_index[1]
    zeros = jnp.zeros((N, H), jnp.float32)

    def gin(h, Wa, ba, Wb, bb):
        p = _sc_aggregate(h, src, dst, zeros)
        return _tc_layer(h, p, Wa, ba.reshape(1, H), Wb, bb.reshape(1, H))

    x1 = gin(x, W1a, b1a, W1b, b1b)
    x2 = gin(x1, W2a, b2a, W2b, b2b)
    x3 = gin(x2, W3a, b3a, W3b, b3b)

    wtc_p = jnp.zeros((H, H), jnp.float32).at[:, :1].set(Wtc)
    btc_p = jnp.zeros((1, H), jnp.float32).at[0, :1].set(btc)
    wc_p = jnp.zeros((H, H), jnp.float32).at[:, :C].set(Wc)
    bc_p = jnp.zeros((1, H), jnp.float32).at[0, :C].set(bc)

    out = _tc_attn(x3, Wta, bta.reshape(1, H), Wtb, btb.reshape(1, H),
                   wtc_p, btc_p, Wr, br.reshape(1, H), wc_p, bc_p)
    return out[:, :C]
